# baseline (device time: 26040 ns/iter reference)
import jax
import jax.numpy as jnp
from jax import lax
from jax.experimental import pallas as pl
from jax.experimental.pallas import tpu as pltpu

N_DEV = 4


def _gelu(y):
    c = 0.7978845608028654
    return 0.5 * y * (1.0 + jnp.tanh(c * (y + 0.044715 * y * y * y)))


def kernel(x, w_mat):
    m_per, k = x.shape
    _, n_per = w_mat.shape
    h = m_per // 2

    def body(x_hbm, w_hbm, out_hbm, x_v, w_v, buf_l, buf_r, buf_o, res_v,
             send_sems, recv_sems, local_sems):
        my_pos = lax.axis_index("i")
        left = lax.rem(my_pos + (N_DEV - 1), N_DEV)
        right = lax.rem(my_pos + 1, N_DEV)

        def copy(src, dst, sem_idx, target):
            return pltpu.make_async_remote_copy(
                src_ref=src, dst_ref=dst,
                send_sem=send_sems.at[sem_idx], recv_sem=recv_sems.at[sem_idx],
                device_id=(target,), device_id_type=pl.DeviceIdType.MESH,
            )

        def store_chunk(slot, origin):
            cp = pltpu.make_async_copy(
                res_v.at[slot],
                out_hbm.at[pl.ds(origin * m_per, m_per), :],
                local_sems.at[2 + slot],
            )
            cp.start()
            return cp

        cp_x = pltpu.make_async_copy(x_hbm, x_v, local_sems.at[0])
        cp_w = pltpu.make_async_copy(w_hbm, w_v, local_sems.at[1])
        cp_x.start()
        cp_w.start()

        barrier_sem = pltpu.get_barrier_semaphore()
        for nbr in [left, right]:
            pl.semaphore_signal(
                barrier_sem, inc=1,
                device_id=(nbr,), device_id_type=pl.DeviceIdType.MESH,
            )
        pl.semaphore_wait(barrier_sem, 2)

        cp_x.wait()
        r1a = copy(x_v.at[pl.ds(0, h), :], buf_l.at[pl.ds(0, h), :], 0, right)
        r1b = copy(x_v.at[pl.ds(h, h), :], buf_l.at[pl.ds(h, h), :], 1, right)
        l1a = copy(x_v.at[pl.ds(h, h), :], buf_r.at[pl.ds(h, h), :], 2, left)
        l1b = copy(x_v.at[pl.ds(0, h), :], buf_r.at[pl.ds(0, h), :], 3, left)
        r1a.start()
        l1a.start()
        r1b.start()
        l1b.start()

        cp_w.wait()
        res_v[0] = _gelu(
            jnp.dot(x_v[...], w_v[...], preferred_element_type=jnp.float32)
        )
        st0 = store_chunk(0, my_pos)

        r1a.wait_recv()
        r2 = copy(buf_l.at[pl.ds(0, h), :], buf_o.at[pl.ds(0, h), :], 4, right)
        r2.start()
        l1a.wait_recv()
        l2 = copy(buf_r.at[pl.ds(h, h), :], buf_o.at[pl.ds(h, h), :], 5, left)
        l2.start()

        r1b.wait_recv()
        res_v[1] = _gelu(
            jnp.dot(buf_l[...], w_v[...], preferred_element_type=jnp.float32)
        )
        st1 = store_chunk(1, lax.rem(my_pos + (N_DEV - 1), N_DEV))

        l1b.wait_recv()
        res_v[2] = _gelu(
            jnp.dot(buf_r[...], w_v[...], preferred_element_type=jnp.float32)
        )
        st2 = store_chunk(2, lax.rem(my_pos + 1, N_DEV))

        r2.wait_recv()
        l2.wait_recv()
        res_v[3] = _gelu(
            jnp.dot(buf_o[...], w_v[...], preferred_element_type=jnp.float32)
        )
        st3 = store_chunk(3, lax.rem(my_pos + 2, N_DEV))

        st0.wait()
        st1.wait()
        st2.wait()
        st3.wait()
        for rdma in (r1a, r1b, l1a, l1b, r2, l2):
            rdma.wait_send()

    return pl.pallas_call(
        body,
        out_shape=jax.ShapeDtypeStruct((N_DEV * m_per, n_per), jnp.float32),
        in_specs=[
            pl.BlockSpec(memory_space=pl.ANY),
            pl.BlockSpec(memory_space=pl.ANY),
        ],
        out_specs=pl.BlockSpec(memory_space=pl.ANY),
        scratch_shapes=[
            pltpu.VMEM((m_per, k), jnp.float32),
            pltpu.VMEM((k, n_per), jnp.float32),
            pltpu.VMEM((m_per, k), jnp.float32),
            pltpu.VMEM((m_per, k), jnp.float32),
            pltpu.VMEM((m_per, k), jnp.float32),
            pltpu.VMEM((N_DEV, m_per, n_per), jnp.float32),
            pltpu.SemaphoreType.DMA((6,)),
            pltpu.SemaphoreType.DMA((6,)),
            pltpu.SemaphoreType.DMA((6,)),
        ],
        compiler_params=pltpu.CompilerParams(collective_id=0),
    )(x, w_mat)


# device time: 16270 ns/iter; 1.6005x vs baseline; 1.6005x over previous
import jax
import jax.numpy as jnp
from jax import lax
from jax.experimental import pallas as pl
from jax.experimental.pallas import tpu as pltpu

N_DEV = 4


def _gelu(y):
    c = 0.7978845608028654
    return 0.5 * y * (1.0 + jnp.tanh(c * (y + 0.044715 * y * y * y)))


def kernel(x, w_mat):
    m_per, k = x.shape
    _, n_per = w_mat.shape
    h = m_per // 2

    def body(x_ref, w_ref, out_ref, x_bf, buf_l, buf_r, buf_o,
             send_sems, recv_sems):
        my_pos = lax.axis_index("i")
        left = lax.rem(my_pos + (N_DEV - 1), N_DEV)
        right = lax.rem(my_pos + 1, N_DEV)

        def copy(src, dst, sem_idx, target):
            return pltpu.make_async_remote_copy(
                src_ref=src, dst_ref=dst,
                send_sem=send_sems.at[sem_idx], recv_sem=recv_sems.at[sem_idx],
                device_id=(target,), device_id_type=pl.DeviceIdType.MESH,
            )

        x_bf[...] = x_ref[...].astype(jnp.bfloat16)

        barrier_sem = pltpu.get_barrier_semaphore()
        for nbr in [left, right]:
            pl.semaphore_signal(
                barrier_sem, inc=1,
                device_id=(nbr,), device_id_type=pl.DeviceIdType.MESH,
            )
        pl.semaphore_wait(barrier_sem, 2)

        r1a = copy(x_bf.at[pl.ds(0, h), :], buf_l.at[pl.ds(0, h), :], 0, right)
        r1b = copy(x_bf.at[pl.ds(h, h), :], buf_l.at[pl.ds(h, h), :], 1, right)
        l1a = copy(x_bf.at[pl.ds(h, h), :], buf_r.at[pl.ds(h, h), :], 2, left)
        l1b = copy(x_bf.at[pl.ds(0, h), :], buf_r.at[pl.ds(0, h), :], 3, left)
        r1a.start()
        l1a.start()
        r1b.start()
        l1b.start()

        out_ref[pl.ds(my_pos * m_per, m_per), :] = _gelu(
            jnp.dot(x_ref[...], w_ref[...], preferred_element_type=jnp.float32)
        )

        r1a.wait_recv()
        r2 = copy(buf_l.at[pl.ds(0, h), :], buf_o.at[pl.ds(0, h), :], 4, right)
        r2.start()
        l1a.wait_recv()
        l2 = copy(buf_r.at[pl.ds(h, h), :], buf_o.at[pl.ds(h, h), :], 5, left)
        l2.start()

        r1b.wait_recv()
        origin = lax.rem(my_pos + (N_DEV - 1), N_DEV)
        out_ref[pl.ds(origin * m_per, m_per), :] = _gelu(
            jnp.dot(buf_l[...], w_ref[...], preferred_element_type=jnp.float32)
        )
        l1b.wait_recv()
        origin = lax.rem(my_pos + 1, N_DEV)
        out_ref[pl.ds(origin * m_per, m_per), :] = _gelu(
            jnp.dot(buf_r[...], w_ref[...], preferred_element_type=jnp.float32)
        )

        r2.wait_recv()
        l2.wait_recv()
        origin = lax.rem(my_pos + 2, N_DEV)
        out_ref[pl.ds(origin * m_per, m_per), :] = _gelu(
            jnp.dot(buf_o[...], w_ref[...], preferred_element_type=jnp.float32)
        )

        for rdma in (r1a, r1b, l1a, l1b, r2, l2):
            rdma.wait_send()

    return pl.pallas_call(
        body,
        out_shape=jax.ShapeDtypeStruct((N_DEV * m_per, n_per), jnp.float32),
        in_specs=[
            pl.BlockSpec(memory_space=pltpu.VMEM),
            pl.BlockSpec(memory_space=pltpu.VMEM),
        ],
        out_specs=pl.BlockSpec(memory_space=pltpu.VMEM),
        scratch_shapes=[
            pltpu.VMEM((m_per, k), jnp.bfloat16),
            pltpu.VMEM((m_per, k), jnp.bfloat16),
            pltpu.VMEM((m_per, k), jnp.bfloat16),
            pltpu.VMEM((m_per, k), jnp.bfloat16),
            pltpu.SemaphoreType.DMA((6,)),
            pltpu.SemaphoreType.DMA((6,)),
        ],
        compiler_params=pltpu.CompilerParams(collective_id=0),
    )(x, w_mat)
